# baseline (speedup 1.0000x reference)
import jax
import jax.numpy as jnp
from jax import lax
from jax.experimental import pallas as pl
from jax.experimental.pallas import tpu as pltpu
from jax.experimental.pallas import tpu_sc as plsc

N = 10000
E = 320000
R = 8
D = 128
NPAD = 10240
NRP = NPAD * R
NC = 2
NS = 16
NW = NC * NS
EP = 327680
EPW = EP // NW
SUB = 64
NSUB = EPW // SUB
MROWS = EP // SUB
MEGA = 16
NMEGA = NSUB // MEGA


def _sc_mesh():
    return plsc.VectorSubcoreMesh(
        core_axis_name="c", subcore_axis_name="s", num_cores=NC, num_subcores=NS
    )


def _sc_params():
    return pltpu.CompilerParams(needs_layout_passes=False)


def _sc_prepass(dstp, typp, zeros_hbm, ones_hbm, cnt2_hbm,
                segb, typb, onesb, cb, cnt_acc, sem):
    c = lax.axis_index("c")
    s = lax.axis_index("s")
    w = c * NS + s
    pltpu.sync_copy(zeros_hbm, cb)
    pltpu.sync_copy(cb, cnt_acc.at[pl.ds(s * (NRP // NS), NRP // NS)])
    pltpu.sync_copy(ones_hbm, onesb)
    pltpu.sync_copy(dstp.at[pl.ds(w * NSUB, NSUB)], segb)
    pltpu.sync_copy(typp.at[pl.ds(w * NSUB, NSUB)], typb)
    plsc.subcore_barrier()

    @plsc.parallel_loop(0, NSUB, unroll=2)
    def mkseg(r):
        for c8 in range(SUB // 16):
            sl = pl.ds(c8 * 16, 16)
            segb[r, sl] = segb[r, sl] * R + typb[r, sl]

    def fire8(k, carry):
        for b in range(8):
            pltpu.async_copy(onesb, cnt_acc.at[segb.at[k * 8 + b]], sem,
                             add=True)
        for b in range(8):
            pltpu.make_async_copy(onesb, cnt_acc.at[segb.at[k * 8 + b]],
                                  sem).wait()
        return carry

    lax.fori_loop(0, NSUB // 8, fire8, 0)
    plsc.subcore_barrier()
    pltpu.sync_copy(cnt_acc.at[pl.ds(s * (NRP // NS), NRP // NS)], cb)
    pltpu.sync_copy(cb, cnt2_hbm.at[pl.ds(c * NRP + s * (NRP // NS),
                                          NRP // NS)])


def _run_prepass(dstp, typp):
    zeros = jnp.zeros((NRP // NS,), jnp.float32)
    ones = jnp.ones((SUB,), jnp.float32)
    fn = pl.kernel(
        _sc_prepass,
        out_type=jax.ShapeDtypeStruct((NC * NRP,), jnp.float32),
        mesh=_sc_mesh(),
        scratch_types=[
            pltpu.VMEM((NSUB, SUB), jnp.int32),
            pltpu.VMEM((NSUB, SUB), jnp.int32),
            pltpu.VMEM((SUB,), jnp.float32),
            pltpu.VMEM((NRP // NS,), jnp.float32),
            pltpu.VMEM_SHARED((NRP,), jnp.float32),
            pltpu.SemaphoreType.DMA,
        ],
        compiler_params=_sc_params(),
    )
    return fn(dstp, typp, zeros, ones)


def _make_edge_pass(use_rc):
    def body(table_hbm, srcp, typp, dstp, w_hbm, zrows_hbm, part_hbm,
             *scratch):
        if use_rc:
            (ab, bb, cbuf, wb, rcb, g0, g1, s0, s1, rcsp, acc,
             sem0, sem1, ssem0, ssem1) = scratch
        else:
            (ab, cbuf, wsb, wb, g0, g1, s0, s1, acc,
             sem0, sem1, ssem0, ssem1) = scratch
        c = lax.axis_index("c")
        s = lax.axis_index("s")
        w = c * NS + s
        rows_per_tile = NPAD // NS
        pltpu.sync_copy(zrows_hbm, g0)
        for k in range(rows_per_tile // SUB):
            pltpu.sync_copy(g0, acc.at[pl.ds(s * rows_per_tile + k * SUB,
                                             SUB)])
        if use_rc:
            quart = NRP // NS // 4
            for h in range(4):
                off = s * (NRP // NS) + h * quart
                pltpu.sync_copy(w_hbm.at[pl.ds(off, quart)], rcb)
                pltpu.sync_copy(rcb, rcsp.at[pl.ds(off, quart)])
        plsc.subcore_barrier()

        def gather(j, gbuf, sem):
            pltpu.async_copy(table_hbm.at[ab.at[j]], gbuf, sem)

        def gwait(j, gbuf, sem):
            pltpu.make_async_copy(table_hbm.at[ab.at[j]], gbuf, sem).wait()

        def scat(j, sbuf, sem):
            pltpu.async_copy(sbuf, acc.at[cbuf.at[j]], sem, add=True)

        def swait(j, sbuf, sem):
            pltpu.make_async_copy(sbuf, acc.at[cbuf.at[j]], sem).wait()

        def scale(j, gbuf, sbuf):
            if use_rc:
                pltpu.sync_copy(rcsp.at[bb.at[j]], wb)

            @plsc.parallel_loop(0, SUB, unroll=2)
            def _scale(r):
                r16 = jnp.full((16,), r, jnp.int32)
                if use_rc:
                    wv = plsc.load_gather(wb, [r16])
                else:
                    wv = plsc.load_gather(
                        wsb, [jnp.full((16,), j, jnp.int32), r16])
                for c8 in range(D // 16):
                    sl = pl.ds(c8 * 16, 16)
                    sbuf[r, sl] = gbuf[r, sl] * wv

        asubs = jnp.minimum(NSUB, jnp.maximum(0, (E // SUB) - w * NSUB))
        for m in range(NMEGA):
            pairs = jnp.clip((asubs - m * MEGA) // 2, 0, MEGA // 2)

            @pl.when(pairs > 0)
            def _mega():
                md = pl.ds(w * NSUB + m * MEGA, MEGA)
                pltpu.sync_copy(srcp.at[md], ab)
                pltpu.sync_copy(dstp.at[md], cbuf)
                if use_rc:
                    pltpu.sync_copy(typp.at[md], bb)

                    @plsc.parallel_loop(0, MEGA, unroll=2)
                    def mkidx(r):
                        for c8 in range(SUB // 16):
                            sl = pl.ds(c8 * 16, 16)
                            tv = bb[r, sl]
                            ab[r, sl] = ab[r, sl] * R + tv
                            bb[r, sl] = cbuf[r, sl] * R + tv
                else:
                    pltpu.sync_copy(w_hbm.at[md], wsb)

                gather(0, g0, sem0)
                gather(1, g1, sem1)

                def pair(k, carry):
                    j0 = 2 * k
                    gwait(j0, g0, sem0)

                    @pl.when(k > 0)
                    def _():
                        swait(j0 - 2, s0, ssem0)

                    scale(j0, g0, s0)

                    @pl.when(j0 + 2 < 2 * pairs)
                    def _():
                        gather(j0 + 2, g0, sem0)

                    scat(j0, s0, ssem0)
                    gwait(j0 + 1, g1, sem1)

                    @pl.when(k > 0)
                    def _():
                        swait(j0 - 1, s1, ssem1)

                    scale(j0 + 1, g1, s1)

                    @pl.when(j0 + 3 < 2 * pairs)
                    def _():
                        gather(j0 + 3, g1, sem1)

                    scat(j0 + 1, s1, ssem1)
                    return carry

                lax.fori_loop(0, pairs, pair, 0)
                swait(2 * pairs - 2, s0, ssem0)
                swait(2 * pairs - 1, s1, ssem1)
        plsc.subcore_barrier()
        for k in range(rows_per_tile // SUB):
            off = s * rows_per_tile + k * SUB
            pltpu.sync_copy(acc.at[pl.ds(off, SUB)], g0)
            pltpu.sync_copy(g0, part_hbm.at[pl.ds(c * NPAD + off, SUB)])

    return body


def _run_edge_pass(table, srcp, typp, dstp, w, use_rc):
    zrows = jnp.zeros((SUB, D), jnp.float32)
    scratch = [pltpu.VMEM((MEGA, SUB), jnp.int32)]
    if use_rc:
        scratch.append(pltpu.VMEM((MEGA, SUB), jnp.int32))
    scratch.append(pltpu.VMEM((MEGA, SUB), jnp.int32))
    if use_rc:
        scratch.append(pltpu.VMEM((SUB,), jnp.float32))
        scratch.append(pltpu.VMEM((NRP // NS // 4,), jnp.float32))
    else:
        scratch.append(pltpu.VMEM((MEGA, SUB), jnp.float32))
        scratch.append(pltpu.VMEM((SUB,), jnp.float32))
    scratch.append(pltpu.VMEM((SUB, D), jnp.float32))
    scratch.append(pltpu.VMEM((SUB, D), jnp.float32))
    scratch.append(pltpu.VMEM((SUB, D), jnp.float32))
    scratch.append(pltpu.VMEM((SUB, D), jnp.float32))
    if use_rc:
        scratch.append(pltpu.VMEM_SHARED((NRP,), jnp.float32))
    scratch.append(pltpu.VMEM_SHARED((NPAD, D), jnp.float32))
    scratch.append(pltpu.SemaphoreType.DMA)
    scratch.append(pltpu.SemaphoreType.DMA)
    scratch.append(pltpu.SemaphoreType.DMA)
    scratch.append(pltpu.SemaphoreType.DMA)
    fn = pl.kernel(
        _make_edge_pass(use_rc),
        out_type=jax.ShapeDtypeStruct((NC * NPAD, D), jnp.float32),
        mesh=_sc_mesh(),
        scratch_types=scratch,
        compiler_params=_sc_params(),
    )
    return fn(table, srcp, typp, dstp, w, zrows).reshape(NC, NPAD, D)


def _mm_kernel(a_ref, b_ref, o_ref):
    o_ref[...] = jnp.dot(a_ref[...], b_ref[...],
                         preferred_element_type=jnp.float32)


def _tc_wflat(comp, basis_flat):
    return pl.pallas_call(
        _mm_kernel,
        grid=(8,),
        in_specs=[
            pl.BlockSpec((R, 30), lambda j: (0, 0)),
            pl.BlockSpec((30, 2048), lambda j: (0, j)),
        ],
        out_specs=pl.BlockSpec((R, 2048), lambda j: (0, j)),
        out_shape=jax.ShapeDtypeStruct((R, D * D), jnp.float32),
    )(comp, basis_flat)


def _tc_table1(x, wall):
    return pl.pallas_call(
        _mm_kernel,
        grid=(5,),
        in_specs=[
            pl.BlockSpec((2000, D), lambda i: (i, 0)),
            pl.BlockSpec((D, R * D), lambda i: (0, 0)),
        ],
        out_specs=pl.BlockSpec((2000, R * D), lambda i: (i, 0)),
        out_shape=jax.ShapeDtypeStruct((N, R * D), jnp.float32),
    )(x, wall)


def _rc_kernel(cnt_ref, rc_ref):
    c = cnt_ref[0] + cnt_ref[1]
    rc_ref[...] = 1.0 / jnp.maximum(c, 1.0)


def _tc_rc(cnt2):
    return pl.pallas_call(
        _rc_kernel,
        out_shape=jax.ShapeDtypeStruct((NRP // D, D), jnp.float32),
    )(cnt2)


def _gate1_kernel(x_ref, p_ref, root_ref, b1_ref, gw_ref, gb_ref, o_ref):
    x = x_ref[...]
    hg = (p_ref[0] + p_ref[1]
          + jnp.dot(x, root_ref[...], preferred_element_type=jnp.float32)
          + b1_ref[...])
    z = (jnp.dot(x, gw_ref[:D, :], preferred_element_type=jnp.float32)
         + jnp.dot(hg, gw_ref[D:, :], preferred_element_type=jnp.float32)
         + gb_ref[...])
    a = jax.nn.sigmoid(z)
    o_ref[...] = a * hg + (1.0 - a) * x


def _tc_gate1(x, parts, root1, bias1, g1w, g1b):
    return pl.pallas_call(
        _gate1_kernel,
        grid=(5,),
        in_specs=[
            pl.BlockSpec((2000, D), lambda i: (i, 0)),
            pl.BlockSpec((NC, 2000, D), lambda i: (0, i, 0)),
            pl.BlockSpec((D, D), lambda i: (0, 0)),
            pl.BlockSpec((1, D), lambda i: (0, 0)),
            pl.BlockSpec((2 * D, D), lambda i: (0, 0)),
            pl.BlockSpec((1, D), lambda i: (0, 0)),
        ],
        out_specs=pl.BlockSpec((2000, D), lambda i: (i, 0)),
        out_shape=jax.ShapeDtypeStruct((N, D), jnp.float32),
    )(x, parts, root1, bias1, g1w, g1b)


def _gate2_kernel(h1_ref, a_ref, wrel_ref, brel_ref, wroot_ref, gw_ref,
                  gb_ref, o_ref):
    h1 = h1_ref[...]
    agg = a_ref[0] + a_ref[1]
    hg = (jnp.dot(agg, wrel_ref[...], preferred_element_type=jnp.float32)
          + brel_ref[...]
          + jnp.dot(h1, wroot_ref[...], preferred_element_type=jnp.float32))
    z = (jnp.dot(h1, gw_ref[:D, :], preferred_element_type=jnp.float32)
         + jnp.dot(hg, gw_ref[D:, :], preferred_element_type=jnp.float32)
         + gb_ref[...])
    a = jax.nn.sigmoid(z)
    o_ref[...] = a * hg + (1.0 - a) * h1


def _tc_gate2(h1, parts, wrel, brel, wroot, g2w, g2b):
    return pl.pallas_call(
        _gate2_kernel,
        grid=(5,),
        in_specs=[
            pl.BlockSpec((2000, D), lambda i: (i, 0)),
            pl.BlockSpec((NC, 2000, D), lambda i: (0, i, 0)),
            pl.BlockSpec((D, D), lambda i: (0, 0)),
            pl.BlockSpec((1, D), lambda i: (0, 0)),
            pl.BlockSpec((D, D), lambda i: (0, 0)),
            pl.BlockSpec((2 * D, D), lambda i: (0, 0)),
            pl.BlockSpec((1, D), lambda i: (0, 0)),
        ],
        out_specs=pl.BlockSpec((2000, D), lambda i: (i, 0)),
        out_shape=jax.ShapeDtypeStruct((N, D), jnp.float32),
    )(h1, parts, wrel, brel, wroot, g2w, g2b)


def kernel(node_features, edge_index, edge_norm, edge_type, basis, comp,
           root1, bias1, gc_w_rel, gc_b_rel, gc_w_root,
           gate1_w, gate1_b, gate2_w, gate2_b):
    npad_e = EP - E
    srcp = jnp.concatenate(
        [edge_index[0], jnp.zeros((npad_e,), jnp.int32)]).reshape(MROWS, SUB)
    dstp = jnp.concatenate(
        [edge_index[1],
         N + (jnp.arange(npad_e, dtype=jnp.int32) % (NPAD - N))]
    ).reshape(MROWS, SUB)
    typp = jnp.concatenate(
        [edge_type, jnp.zeros((npad_e,), jnp.int32)]).reshape(MROWS, SUB)
    normp = jnp.concatenate(
        [edge_norm, jnp.zeros((npad_e,), jnp.float32)]).reshape(MROWS, SUB)

    cnt2 = _run_prepass(dstp, typp)

    wflat = _tc_wflat(comp, basis.reshape(30, D * D))
    wall = wflat.reshape(R, D, D).transpose(1, 0, 2).reshape(D, R * D)
    table1 = _tc_table1(node_features, wall).reshape(N * R, D)
    rc = _tc_rc(cnt2.reshape(NC, NRP // D, D)).reshape(NRP)

    p1 = _run_edge_pass(table1, srcp, typp, dstp, rc, use_rc=True)

    h1 = _tc_gate1(node_features, p1, root1, bias1.reshape(1, D),
                   gate1_w, gate1_b.reshape(1, D))

    p2 = _run_edge_pass(h1, srcp, typp, dstp, normp, use_rc=False)

    h2 = _tc_gate2(h1, p2, gc_w_rel, gc_b_rel.reshape(1, D), gc_w_root,
                   gate2_w, gate2_b.reshape(1, D))
    return h2


# speedup vs baseline: 1.0433x; 1.0433x over previous
import jax
import jax.numpy as jnp
from jax import lax
from jax.experimental import pallas as pl
from jax.experimental.pallas import tpu as pltpu
from jax.experimental.pallas import tpu_sc as plsc

N = 10000
E = 320000
R = 8
D = 128
NPAD = 10240
NRP = NPAD * R
NC = 2
NS = 16
NW = NC * NS
EP = 327680
EPW = EP // NW
SUB = 64
NSUB = EPW // SUB
MROWS = EP // SUB
MEGA = 16
NMEGA = NSUB // MEGA


def _sc_mesh():
    return plsc.VectorSubcoreMesh(
        core_axis_name="c", subcore_axis_name="s", num_cores=NC, num_subcores=NS
    )


def _sc_params():
    return pltpu.CompilerParams(needs_layout_passes=False)


def _sc_prepass(dstp, typp, zeros_hbm, ones_hbm, cnt2_hbm,
                segb, typb, onesb, cb, cnt_acc, sem):
    c = lax.axis_index("c")
    s = lax.axis_index("s")
    w = c * NS + s
    pltpu.sync_copy(zeros_hbm, cb)
    pltpu.sync_copy(cb, cnt_acc.at[pl.ds(s * (NRP // NS), NRP // NS)])
    pltpu.sync_copy(ones_hbm, onesb)
    pltpu.sync_copy(dstp.at[pl.ds(w * NSUB, NSUB)], segb)
    pltpu.sync_copy(typp.at[pl.ds(w * NSUB, NSUB)], typb)
    plsc.subcore_barrier()

    @plsc.parallel_loop(0, NSUB, unroll=2)
    def mkseg(r):
        for c8 in range(SUB // 16):
            sl = pl.ds(c8 * 16, 16)
            segb[r, sl] = segb[r, sl] * R + typb[r, sl]

    def fire8(k, carry):
        for b in range(8):
            pltpu.async_copy(onesb, cnt_acc.at[segb.at[k * 8 + b]], sem,
                             add=True)
        for b in range(8):
            pltpu.make_async_copy(onesb, cnt_acc.at[segb.at[k * 8 + b]],
                                  sem).wait()
        return carry

    lax.fori_loop(0, NSUB // 8, fire8, 0)
    plsc.subcore_barrier()
    pltpu.sync_copy(cnt_acc.at[pl.ds(s * (NRP // NS), NRP // NS)], cb)
    pltpu.sync_copy(cb, cnt2_hbm.at[pl.ds(c * NRP + s * (NRP // NS),
                                          NRP // NS)])


def _run_prepass(dstp, typp):
    zeros = jnp.zeros((NRP // NS,), jnp.float32)
    ones = jnp.ones((SUB,), jnp.float32)
    fn = pl.kernel(
        _sc_prepass,
        out_type=jax.ShapeDtypeStruct((NC * NRP,), jnp.float32),
        mesh=_sc_mesh(),
        scratch_types=[
            pltpu.VMEM((NSUB, SUB), jnp.int32),
            pltpu.VMEM((NSUB, SUB), jnp.int32),
            pltpu.VMEM((SUB,), jnp.float32),
            pltpu.VMEM((NRP // NS,), jnp.float32),
            pltpu.VMEM_SHARED((NRP,), jnp.float32),
            pltpu.SemaphoreType.DMA,
        ],
        compiler_params=_sc_params(),
    )
    return fn(dstp, typp, zeros, ones)


def _make_edge_pass(use_rc, mega):
    nmega = NSUB // mega

    def body(table_hbm, srcp, typp, dstp, w_hbm, zrows_hbm, part_hbm,
             *scratch):
        if use_rc:
            (ab, bb, cbuf, wsb, rcb, g0, g1, s0, s1, rcsp, acc,
             sem0, sem1, ssem0, ssem1, wsem) = scratch
        else:
            (ab, cbuf, wsb, g0, g1, s0, s1, acc,
             sem0, sem1, ssem0, ssem1) = scratch
        c = lax.axis_index("c")
        s = lax.axis_index("s")
        w = c * NS + s
        rows_per_tile = NPAD // NS
        pltpu.sync_copy(zrows_hbm, g0)
        for k in range(rows_per_tile // SUB):
            pltpu.sync_copy(g0, acc.at[pl.ds(s * rows_per_tile + k * SUB,
                                             SUB)])
        if use_rc:
            quart = NRP // NS // 4
            for h in range(4):
                off = s * (NRP // NS) + h * quart
                pltpu.sync_copy(w_hbm.at[pl.ds(off, quart)], rcb)
                pltpu.sync_copy(rcb, rcsp.at[pl.ds(off, quart)])
        plsc.subcore_barrier()

        def gather(j, gbuf, sem):
            pltpu.async_copy(table_hbm.at[ab.at[j]], gbuf, sem)

        def gwait(j, gbuf, sem):
            pltpu.make_async_copy(table_hbm.at[ab.at[j]], gbuf, sem).wait()

        def scat(j, sbuf, sem):
            pltpu.async_copy(sbuf, acc.at[cbuf.at[j]], sem, add=True)

        def swait(j, sbuf, sem):
            pltpu.make_async_copy(sbuf, acc.at[cbuf.at[j]], sem).wait()

        def scale(j, gbuf, sbuf):
            @plsc.parallel_loop(0, SUB, unroll=2)
            def _scale(r):
                wv = plsc.load_gather(
                    wsb, [jnp.full((16,), j, jnp.int32),
                          jnp.full((16,), r, jnp.int32)])
                for c8 in range(D // 16):
                    sl = pl.ds(c8 * 16, 16)
                    sbuf[r, sl] = gbuf[r, sl] * wv

        asubs = jnp.minimum(NSUB, jnp.maximum(0, (E // SUB) - w * NSUB))
        for m in range(nmega):
            pairs = jnp.clip((asubs - m * mega) // 2, 0, mega // 2)

            @pl.when(pairs > 0)
            def _mega():
                md = pl.ds(w * NSUB + m * mega, mega)
                pltpu.sync_copy(srcp.at[md], ab)
                pltpu.sync_copy(dstp.at[md], cbuf)
                if use_rc:
                    pltpu.sync_copy(typp.at[md], bb)

                    @plsc.parallel_loop(0, mega, unroll=2)
                    def mkidx(r):
                        for c8 in range(SUB // 16):
                            sl = pl.ds(c8 * 16, 16)
                            tv = bb[r, sl]
                            ab[r, sl] = ab[r, sl] * R + tv
                            bb[r, sl] = cbuf[r, sl] * R + tv

                    for r in range(mega):
                        pltpu.async_copy(rcsp.at[bb.at[r]], wsb.at[r], wsem)
                    for r in range(mega):
                        pltpu.make_async_copy(rcsp.at[bb.at[r]], wsb.at[r],
                                              wsem).wait()
                else:
                    pltpu.sync_copy(w_hbm.at[md], wsb)

                gather(0, g0, sem0)
                gather(1, g1, sem1)

                def pair(k, carry):
                    j0 = 2 * k
                    gwait(j0, g0, sem0)

                    @pl.when(k > 0)
                    def _():
                        swait(j0 - 2, s0, ssem0)

                    scale(j0, g0, s0)

                    @pl.when(j0 + 2 < 2 * pairs)
                    def _():
                        gather(j0 + 2, g0, sem0)

                    scat(j0, s0, ssem0)
                    gwait(j0 + 1, g1, sem1)

                    @pl.when(k > 0)
                    def _():
                        swait(j0 - 1, s1, ssem1)

                    scale(j0 + 1, g1, s1)

                    @pl.when(j0 + 3 < 2 * pairs)
                    def _():
                        gather(j0 + 3, g1, sem1)

                    scat(j0 + 1, s1, ssem1)
                    return carry

                lax.fori_loop(0, pairs, pair, 0)
                swait(2 * pairs - 2, s0, ssem0)
                swait(2 * pairs - 1, s1, ssem1)
        plsc.subcore_barrier()
        for k in range(rows_per_tile // SUB):
            off = s * rows_per_tile + k * SUB
            pltpu.sync_copy(acc.at[pl.ds(off, SUB)], g0)
            pltpu.sync_copy(g0, part_hbm.at[pl.ds(c * NPAD + off, SUB)])

    return body


def _run_edge_pass(table, srcp, typp, dstp, w, use_rc):
    mega = 16 if use_rc else 32
    zrows = jnp.zeros((SUB, D), jnp.float32)
    scratch = [pltpu.VMEM((mega, SUB), jnp.int32)]
    if use_rc:
        scratch.append(pltpu.VMEM((mega, SUB), jnp.int32))
    scratch.append(pltpu.VMEM((mega, SUB), jnp.int32))
    scratch.append(pltpu.VMEM((mega, SUB), jnp.float32))
    if use_rc:
        scratch.append(pltpu.VMEM((NRP // NS // 4,), jnp.float32))
    scratch.append(pltpu.VMEM((SUB, D), jnp.float32))
    scratch.append(pltpu.VMEM((SUB, D), jnp.float32))
    scratch.append(pltpu.VMEM((SUB, D), jnp.float32))
    scratch.append(pltpu.VMEM((SUB, D), jnp.float32))
    if use_rc:
        scratch.append(pltpu.VMEM_SHARED((NRP,), jnp.float32))
    scratch.append(pltpu.VMEM_SHARED((NPAD, D), jnp.float32))
    scratch.append(pltpu.SemaphoreType.DMA)
    scratch.append(pltpu.SemaphoreType.DMA)
    scratch.append(pltpu.SemaphoreType.DMA)
    scratch.append(pltpu.SemaphoreType.DMA)
    if use_rc:
        scratch.append(pltpu.SemaphoreType.DMA)
    fn = pl.kernel(
        _make_edge_pass(use_rc, mega),
        out_type=jax.ShapeDtypeStruct((NC * NPAD, D), jnp.float32),
        mesh=_sc_mesh(),
        scratch_types=scratch,
        compiler_params=_sc_params(),
    )
    return fn(table, srcp, typp, dstp, w, zrows).reshape(NC, NPAD, D)


def _mm_kernel(a_ref, b_ref, o_ref):
    o_ref[...] = jnp.dot(a_ref[...], b_ref[...],
                         preferred_element_type=jnp.float32)


def _tc_wflat(comp, basis_flat):
    return pl.pallas_call(
        _mm_kernel,
        grid=(8,),
        in_specs=[
            pl.BlockSpec((R, 30), lambda j: (0, 0)),
            pl.BlockSpec((30, 2048), lambda j: (0, j)),
        ],
        out_specs=pl.BlockSpec((R, 2048), lambda j: (0, j)),
        out_shape=jax.ShapeDtypeStruct((R, D * D), jnp.float32),
    )(comp, basis_flat)


def _tc_table1(x, wall):
    return pl.pallas_call(
        _mm_kernel,
        grid=(5,),
        in_specs=[
            pl.BlockSpec((2000, D), lambda i: (i, 0)),
            pl.BlockSpec((D, R * D), lambda i: (0, 0)),
        ],
        out_specs=pl.BlockSpec((2000, R * D), lambda i: (i, 0)),
        out_shape=jax.ShapeDtypeStruct((N, R * D), jnp.float32),
    )(x, wall)


def _rc_kernel(cnt_ref, rc_ref):
    c = cnt_ref[0] + cnt_ref[1]
    rc_ref[...] = 1.0 / jnp.maximum(c, 1.0)


def _tc_rc(cnt2):
    return pl.pallas_call(
        _rc_kernel,
        out_shape=jax.ShapeDtypeStruct((NRP // D, D), jnp.float32),
    )(cnt2)


def _gate1_kernel(x_ref, p_ref, root_ref, b1_ref, gw_ref, gb_ref, o_ref):
    x = x_ref[...]
    hg = (p_ref[0] + p_ref[1]
          + jnp.dot(x, root_ref[...], preferred_element_type=jnp.float32)
          + b1_ref[...])
    z = (jnp.dot(x, gw_ref[:D, :], preferred_element_type=jnp.float32)
         + jnp.dot(hg, gw_ref[D:, :], preferred_element_type=jnp.float32)
         + gb_ref[...])
    a = jax.nn.sigmoid(z)
    o_ref[...] = a * hg + (1.0 - a) * x


def _tc_gate1(x, parts, root1, bias1, g1w, g1b):
    return pl.pallas_call(
        _gate1_kernel,
        grid=(5,),
        in_specs=[
            pl.BlockSpec((2000, D), lambda i: (i, 0)),
            pl.BlockSpec((NC, 2000, D), lambda i: (0, i, 0)),
            pl.BlockSpec((D, D), lambda i: (0, 0)),
            pl.BlockSpec((1, D), lambda i: (0, 0)),
            pl.BlockSpec((2 * D, D), lambda i: (0, 0)),
            pl.BlockSpec((1, D), lambda i: (0, 0)),
        ],
        out_specs=pl.BlockSpec((2000, D), lambda i: (i, 0)),
        out_shape=jax.ShapeDtypeStruct((N, D), jnp.float32),
    )(x, parts, root1, bias1, g1w, g1b)


def _gate2_kernel(h1_ref, a_ref, wrel_ref, brel_ref, wroot_ref, gw_ref,
                  gb_ref, o_ref):
    h1 = h1_ref[...]
    agg = a_ref[0] + a_ref[1]
    hg = (jnp.dot(agg, wrel_ref[...], preferred_element_type=jnp.float32)
          + brel_ref[...]
          + jnp.dot(h1, wroot_ref[...], preferred_element_type=jnp.float32))
    z = (jnp.dot(h1, gw_ref[:D, :], preferred_element_type=jnp.float32)
         + jnp.dot(hg, gw_ref[D:, :], preferred_element_type=jnp.float32)
         + gb_ref[...])
    a = jax.nn.sigmoid(z)
    o_ref[...] = a * hg + (1.0 - a) * h1


def _tc_gate2(h1, parts, wrel, brel, wroot, g2w, g2b):
    return pl.pallas_call(
        _gate2_kernel,
        grid=(5,),
        in_specs=[
            pl.BlockSpec((2000, D), lambda i: (i, 0)),
            pl.BlockSpec((NC, 2000, D), lambda i: (0, i, 0)),
            pl.BlockSpec((D, D), lambda i: (0, 0)),
            pl.BlockSpec((1, D), lambda i: (0, 0)),
            pl.BlockSpec((D, D), lambda i: (0, 0)),
            pl.BlockSpec((2 * D, D), lambda i: (0, 0)),
            pl.BlockSpec((1, D), lambda i: (0, 0)),
        ],
        out_specs=pl.BlockSpec((2000, D), lambda i: (i, 0)),
        out_shape=jax.ShapeDtypeStruct((N, D), jnp.float32),
    )(h1, parts, wrel, brel, wroot, g2w, g2b)


def kernel(node_features, edge_index, edge_norm, edge_type, basis, comp,
           root1, bias1, gc_w_rel, gc_b_rel, gc_w_root,
           gate1_w, gate1_b, gate2_w, gate2_b):
    npad_e = EP - E
    srcp = jnp.concatenate(
        [edge_index[0], jnp.zeros((npad_e,), jnp.int32)]).reshape(MROWS, SUB)
    dstp = jnp.concatenate(
        [edge_index[1],
         N + (jnp.arange(npad_e, dtype=jnp.int32) % (NPAD - N))]
    ).reshape(MROWS, SUB)
    typp = jnp.concatenate(
        [edge_type, jnp.zeros((npad_e,), jnp.int32)]).reshape(MROWS, SUB)
    normp = jnp.concatenate(
        [edge_norm, jnp.zeros((npad_e,), jnp.float32)]).reshape(MROWS, SUB)

    cnt2 = _run_prepass(dstp, typp)

    wflat = _tc_wflat(comp, basis.reshape(30, D * D))
    wall = wflat.reshape(R, D, D).transpose(1, 0, 2).reshape(D, R * D)
    table1 = _tc_table1(node_features, wall).reshape(N * R, D)
    rc = _tc_rc(cnt2.reshape(NC, NRP // D, D)).reshape(NRP)

    p1 = _run_edge_pass(table1, srcp, typp, dstp, rc, use_rc=True)

    h1 = _tc_gate1(node_features, p1, root1, bias1.reshape(1, D),
                   gate1_w, gate1_b.reshape(1, D))

    p2 = _run_edge_pass(h1, srcp, typp, dstp, normp, use_rc=False)

    h2 = _tc_gate2(h1, p2, gc_w_rel, gc_b_rel.reshape(1, D), gc_w_root,
                   gate2_w, gate2_b.reshape(1, D))
    return h2


# speedup vs baseline: 1.0939x; 1.0485x over previous
import jax
import jax.numpy as jnp
from jax import lax
from jax.experimental import pallas as pl
from jax.experimental.pallas import tpu as pltpu
from jax.experimental.pallas import tpu_sc as plsc

N = 10000
E = 320000
R = 8
D = 128
NPAD = 10240
NRP = NPAD * R
NC = 2
NS = 16
NW = NC * NS
EP = 327680
EPW = EP // NW
SUB = 128
NSUB = EPW // SUB
MROWS = EP // SUB
MEGA = 16
NMEGA = NSUB // MEGA


def _sc_mesh():
    return plsc.VectorSubcoreMesh(
        core_axis_name="c", subcore_axis_name="s", num_cores=NC, num_subcores=NS
    )


def _sc_params():
    return pltpu.CompilerParams(needs_layout_passes=False)


def _sc_prepass(dstp, typp, zeros_hbm, ones_hbm, cnt2_hbm,
                segb, typb, onesb, cb, cnt_acc, sem):
    c = lax.axis_index("c")
    s = lax.axis_index("s")
    w = c * NS + s
    pltpu.sync_copy(zeros_hbm, cb)
    pltpu.sync_copy(cb, cnt_acc.at[pl.ds(s * (NRP // NS), NRP // NS)])
    pltpu.sync_copy(ones_hbm, onesb)
    pltpu.sync_copy(dstp.at[pl.ds(w * NSUB, NSUB)], segb)
    pltpu.sync_copy(typp.at[pl.ds(w * NSUB, NSUB)], typb)
    plsc.subcore_barrier()

    @plsc.parallel_loop(0, NSUB, unroll=2)
    def mkseg(r):
        for c8 in range(SUB // 16):
            sl = pl.ds(c8 * 16, 16)
            segb[r, sl] = segb[r, sl] * R + typb[r, sl]

    def fire8(k, carry):
        for b in range(8):
            pltpu.async_copy(onesb, cnt_acc.at[segb.at[k * 8 + b]], sem,
                             add=True)
        for b in range(8):
            pltpu.make_async_copy(onesb, cnt_acc.at[segb.at[k * 8 + b]],
                                  sem).wait()
        return carry

    lax.fori_loop(0, NSUB // 8, fire8, 0)
    plsc.subcore_barrier()
    pltpu.sync_copy(cnt_acc.at[pl.ds(s * (NRP // NS), NRP // NS)], cb)
    pltpu.sync_copy(cb, cnt2_hbm.at[pl.ds(c * NRP + s * (NRP // NS),
                                          NRP // NS)])


def _run_prepass(dstp, typp):
    zeros = jnp.zeros((NRP // NS,), jnp.float32)
    ones = jnp.ones((SUB,), jnp.float32)
    fn = pl.kernel(
        _sc_prepass,
        out_type=jax.ShapeDtypeStruct((NC * NRP,), jnp.float32),
        mesh=_sc_mesh(),
        scratch_types=[
            pltpu.VMEM((NSUB, SUB), jnp.int32),
            pltpu.VMEM((NSUB, SUB), jnp.int32),
            pltpu.VMEM((SUB,), jnp.float32),
            pltpu.VMEM((NRP // NS,), jnp.float32),
            pltpu.VMEM_SHARED((NRP,), jnp.float32),
            pltpu.SemaphoreType.DMA,
        ],
        compiler_params=_sc_params(),
    )
    return fn(dstp, typp, zeros, ones)


def _make_edge_pass(use_rc, mega):
    nmega = NSUB // mega

    def body(table_hbm, srcp, typp, dstp, w_hbm, zrows_hbm, part_hbm,
             *scratch):
        if use_rc:
            (ab, bb, cbuf, wsb, rcb, rows0, rows1, rcsp, acc,
             sem0, sem1, wsem) = scratch
        else:
            (ab, cbuf, wsb, rows0, rows1, acc, sem0, sem1) = scratch
        c = lax.axis_index("c")
        s = lax.axis_index("s")
        w = c * NS + s
        rows_per_tile = NPAD // NS
        pltpu.sync_copy(zrows_hbm, rows0)
        for k in range(rows_per_tile // SUB):
            pltpu.sync_copy(rows0, acc.at[pl.ds(s * rows_per_tile + k * SUB,
                                                SUB)])
        if use_rc:
            quart = NRP // NS // 4
            for h in range(4):
                off = s * (NRP // NS) + h * quart
                pltpu.sync_copy(w_hbm.at[pl.ds(off, quart)], rcb)
                pltpu.sync_copy(rcb, rcsp.at[pl.ds(off, quart)])
        plsc.subcore_barrier()

        def gather(j, rbuf, sem):
            pltpu.async_copy(table_hbm.at[ab.at[j]], rbuf, sem)

        def gwait(j, rbuf, sem):
            pltpu.make_async_copy(table_hbm.at[ab.at[j]], rbuf, sem).wait()

        def process(j, rbuf):
            @plsc.parallel_loop(0, SUB, unroll=2)
            def _scale(r):
                wv = plsc.load_gather(
                    wsb, [jnp.full((16,), j, jnp.int32),
                          jnp.full((16,), r, jnp.int32)])
                for c8 in range(D // 16):
                    sl = pl.ds(c8 * 16, 16)
                    rbuf[r, sl] = rbuf[r, sl] * wv

            pltpu.sync_copy(rbuf, acc.at[cbuf.at[j]], add=True)

        asubs = jnp.minimum(NSUB, jnp.maximum(0, (E // SUB) - w * NSUB))
        for m in range(nmega):
            pairs = jnp.clip((asubs - m * mega) // 2, 0, mega // 2)

            @pl.when(pairs > 0)
            def _mega():
                md = pl.ds(w * NSUB + m * mega, mega)
                pltpu.sync_copy(srcp.at[md], ab)
                pltpu.sync_copy(dstp.at[md], cbuf)
                if use_rc:
                    pltpu.sync_copy(typp.at[md], bb)

                    @plsc.parallel_loop(0, mega, unroll=2)
                    def mkidx(r):
                        for c8 in range(SUB // 16):
                            sl = pl.ds(c8 * 16, 16)
                            tv = bb[r, sl]
                            ab[r, sl] = ab[r, sl] * R + tv
                            bb[r, sl] = cbuf[r, sl] * R + tv

                    for r in range(mega):
                        pltpu.async_copy(rcsp.at[bb.at[r]], wsb.at[r], wsem)
                    for r in range(mega):
                        pltpu.make_async_copy(rcsp.at[bb.at[r]], wsb.at[r],
                                              wsem).wait()
                else:
                    pltpu.sync_copy(w_hbm.at[md], wsb)

                gather(0, rows0, sem0)

                def pair(k, carry):
                    j0 = 2 * k
                    gather(j0 + 1, rows1, sem1)
                    gwait(j0, rows0, sem0)
                    process(j0, rows0)

                    @pl.when(k < pairs - 1)
                    def _():
                        gather(j0 + 2, rows0, sem0)

                    gwait(j0 + 1, rows1, sem1)
                    process(j0 + 1, rows1)
                    return carry

                lax.fori_loop(0, pairs, pair, 0)
        plsc.subcore_barrier()
        for k in range(rows_per_tile // SUB):
            off = s * rows_per_tile + k * SUB
            pltpu.sync_copy(acc.at[pl.ds(off, SUB)], rows0)
            pltpu.sync_copy(rows0, part_hbm.at[pl.ds(c * NPAD + off, SUB)])

    return body


def _run_edge_pass(table, srcp, typp, dstp, w, use_rc):
    mega = 16
    zrows = jnp.zeros((SUB, D), jnp.float32)
    scratch = [pltpu.VMEM((mega, SUB), jnp.int32)]
    if use_rc:
        scratch.append(pltpu.VMEM((mega, SUB), jnp.int32))
    scratch.append(pltpu.VMEM((mega, SUB), jnp.int32))
    scratch.append(pltpu.VMEM((mega, SUB), jnp.float32))
    if use_rc:
        scratch.append(pltpu.VMEM((NRP // NS // 4,), jnp.float32))
    scratch.append(pltpu.VMEM((SUB, D), jnp.float32))
    scratch.append(pltpu.VMEM((SUB, D), jnp.float32))
    if use_rc:
        scratch.append(pltpu.VMEM_SHARED((NRP,), jnp.float32))
    scratch.append(pltpu.VMEM_SHARED((NPAD, D), jnp.float32))
    scratch.append(pltpu.SemaphoreType.DMA)
    scratch.append(pltpu.SemaphoreType.DMA)
    if use_rc:
        scratch.append(pltpu.SemaphoreType.DMA)
    fn = pl.kernel(
        _make_edge_pass(use_rc, mega),
        out_type=jax.ShapeDtypeStruct((NC * NPAD, D), jnp.float32),
        mesh=_sc_mesh(),
        scratch_types=scratch,
        compiler_params=_sc_params(),
    )
    return fn(table, srcp, typp, dstp, w, zrows).reshape(NC, NPAD, D)


def _mm_kernel(a_ref, b_ref, o_ref):
    o_ref[...] = jnp.dot(a_ref[...], b_ref[...],
                         preferred_element_type=jnp.float32)


def _tc_wflat(comp, basis_flat):
    return pl.pallas_call(
        _mm_kernel,
        grid=(8,),
        in_specs=[
            pl.BlockSpec((R, 30), lambda j: (0, 0)),
            pl.BlockSpec((30, 2048), lambda j: (0, j)),
        ],
        out_specs=pl.BlockSpec((R, 2048), lambda j: (0, j)),
        out_shape=jax.ShapeDtypeStruct((R, D * D), jnp.float32),
    )(comp, basis_flat)


def _tc_table1(x, wall):
    return pl.pallas_call(
        _mm_kernel,
        grid=(5,),
        in_specs=[
            pl.BlockSpec((2000, D), lambda i: (i, 0)),
            pl.BlockSpec((D, R * D), lambda i: (0, 0)),
        ],
        out_specs=pl.BlockSpec((2000, R * D), lambda i: (i, 0)),
        out_shape=jax.ShapeDtypeStruct((N, R * D), jnp.float32),
    )(x, wall)


def _rc_kernel(cnt_ref, rc_ref):
    c = cnt_ref[0] + cnt_ref[1]
    rc_ref[...] = 1.0 / jnp.maximum(c, 1.0)


def _tc_rc(cnt2):
    return pl.pallas_call(
        _rc_kernel,
        out_shape=jax.ShapeDtypeStruct((NRP // D, D), jnp.float32),
    )(cnt2)


def _gate1_kernel(x_ref, p_ref, root_ref, b1_ref, gw_ref, gb_ref, o_ref):
    x = x_ref[...]
    hg = (p_ref[0] + p_ref[1]
          + jnp.dot(x, root_ref[...], preferred_element_type=jnp.float32)
          + b1_ref[...])
    z = (jnp.dot(x, gw_ref[:D, :], preferred_element_type=jnp.float32)
         + jnp.dot(hg, gw_ref[D:, :], preferred_element_type=jnp.float32)
         + gb_ref[...])
    a = jax.nn.sigmoid(z)
    o_ref[...] = a * hg + (1.0 - a) * x


def _tc_gate1(x, parts, root1, bias1, g1w, g1b):
    return pl.pallas_call(
        _gate1_kernel,
        grid=(5,),
        in_specs=[
            pl.BlockSpec((2000, D), lambda i: (i, 0)),
            pl.BlockSpec((NC, 2000, D), lambda i: (0, i, 0)),
            pl.BlockSpec((D, D), lambda i: (0, 0)),
            pl.BlockSpec((1, D), lambda i: (0, 0)),
            pl.BlockSpec((2 * D, D), lambda i: (0, 0)),
            pl.BlockSpec((1, D), lambda i: (0, 0)),
        ],
        out_specs=pl.BlockSpec((2000, D), lambda i: (i, 0)),
        out_shape=jax.ShapeDtypeStruct((N, D), jnp.float32),
    )(x, parts, root1, bias1, g1w, g1b)


def _gate2_kernel(h1_ref, a_ref, wrel_ref, brel_ref, wroot_ref, gw_ref,
                  gb_ref, o_ref):
    h1 = h1_ref[...]
    agg = a_ref[0] + a_ref[1]
    hg = (jnp.dot(agg, wrel_ref[...], preferred_element_type=jnp.float32)
          + brel_ref[...]
          + jnp.dot(h1, wroot_ref[...], preferred_element_type=jnp.float32))
    z = (jnp.dot(h1, gw_ref[:D, :], preferred_element_type=jnp.float32)
         + jnp.dot(hg, gw_ref[D:, :], preferred_element_type=jnp.float32)
         + gb_ref[...])
    a = jax.nn.sigmoid(z)
    o_ref[...] = a * hg + (1.0 - a) * h1


def _tc_gate2(h1, parts, wrel, brel, wroot, g2w, g2b):
    return pl.pallas_call(
        _gate2_kernel,
        grid=(5,),
        in_specs=[
            pl.BlockSpec((2000, D), lambda i: (i, 0)),
            pl.BlockSpec((NC, 2000, D), lambda i: (0, i, 0)),
            pl.BlockSpec((D, D), lambda i: (0, 0)),
            pl.BlockSpec((1, D), lambda i: (0, 0)),
            pl.BlockSpec((D, D), lambda i: (0, 0)),
            pl.BlockSpec((2 * D, D), lambda i: (0, 0)),
            pl.BlockSpec((1, D), lambda i: (0, 0)),
        ],
        out_specs=pl.BlockSpec((2000, D), lambda i: (i, 0)),
        out_shape=jax.ShapeDtypeStruct((N, D), jnp.float32),
    )(h1, parts, wrel, brel, wroot, g2w, g2b)


def kernel(node_features, edge_index, edge_norm, edge_type, basis, comp,
           root1, bias1, gc_w_rel, gc_b_rel, gc_w_root,
           gate1_w, gate1_b, gate2_w, gate2_b):
    npad_e = EP - E
    srcp = jnp.concatenate(
        [edge_index[0], jnp.zeros((npad_e,), jnp.int32)]).reshape(MROWS, SUB)
    dstp = jnp.concatenate(
        [edge_index[1],
         N + (jnp.arange(npad_e, dtype=jnp.int32) % (NPAD - N))]
    ).reshape(MROWS, SUB)
    typp = jnp.concatenate(
        [edge_type, jnp.zeros((npad_e,), jnp.int32)]).reshape(MROWS, SUB)
    normp = jnp.concatenate(
        [edge_norm, jnp.zeros((npad_e,), jnp.float32)]).reshape(MROWS, SUB)

    cnt2 = _run_prepass(dstp, typp)

    wflat = _tc_wflat(comp, basis.reshape(30, D * D))
    wall = wflat.reshape(R, D, D).transpose(1, 0, 2).reshape(D, R * D)
    table1 = _tc_table1(node_features, wall).reshape(N * R, D)
    rc = _tc_rc(cnt2.reshape(NC, NRP // D, D)).reshape(NRP)

    p1 = _run_edge_pass(table1, srcp, typp, dstp, rc, use_rc=True)

    h1 = _tc_gate1(node_features, p1, root1, bias1.reshape(1, D),
                   gate1_w, gate1_b.reshape(1, D))

    p2 = _run_edge_pass(h1, srcp, typp, dstp, normp, use_rc=False)

    h2 = _tc_gate2(h1, p2, gc_w_rel, gc_b_rel.reshape(1, D), gc_w_root,
                   gate2_w, gate2_b.reshape(1, D))
    return h2


# speedup vs baseline: 1.1596x; 1.0600x over previous
import jax
import jax.numpy as jnp
from jax import lax
from jax.experimental import pallas as pl
from jax.experimental.pallas import tpu as pltpu
from jax.experimental.pallas import tpu_sc as plsc

N = 10000
E = 320000
R = 8
D = 128
NPAD = 10240
NRP = NPAD * R
NC = 2
NS = 16
NW = NC * NS
EP = 327680
EPW = EP // NW
SUB = 128
NSUB = EPW // SUB
MROWS = EP // SUB
MEGA = 16
NMEGA = NSUB // MEGA


def _sc_mesh():
    return plsc.VectorSubcoreMesh(
        core_axis_name="c", subcore_axis_name="s", num_cores=NC, num_subcores=NS
    )


def _sc_params():
    return pltpu.CompilerParams(needs_layout_passes=False)


def _sc_prepass(dstp, typp, zeros_hbm, ones_hbm, cnt2_hbm,
                segb, typb, onesb, cb, cnt_acc, sem):
    c = lax.axis_index("c")
    s = lax.axis_index("s")
    w = c * NS + s
    pltpu.sync_copy(zeros_hbm, cb)
    pltpu.sync_copy(cb, cnt_acc.at[pl.ds(s * (NRP // NS), NRP // NS)])
    pltpu.sync_copy(ones_hbm, onesb)
    pltpu.sync_copy(dstp.at[pl.ds(w * NSUB, NSUB)], segb)
    pltpu.sync_copy(typp.at[pl.ds(w * NSUB, NSUB)], typb)
    plsc.subcore_barrier()

    @plsc.parallel_loop(0, NSUB, unroll=2)
    def mkseg(r):
        for c8 in range(SUB // 16):
            sl = pl.ds(c8 * 16, 16)
            segb[r, sl] = segb[r, sl] * R + typb[r, sl]

    def fire8(k, carry):
        for b in range(8):
            pltpu.async_copy(onesb, cnt_acc.at[segb.at[k * 8 + b]], sem,
                             add=True)
        for b in range(8):
            pltpu.make_async_copy(onesb, cnt_acc.at[segb.at[k * 8 + b]],
                                  sem).wait()
        return carry

    lax.fori_loop(0, NSUB // 8, fire8, 0)
    plsc.subcore_barrier()
    pltpu.sync_copy(cnt_acc.at[pl.ds(s * (NRP // NS), NRP // NS)], cb)
    pltpu.sync_copy(cb, cnt2_hbm.at[pl.ds(c * NRP + s * (NRP // NS),
                                          NRP // NS)])


def _run_prepass(dstp, typp):
    zeros = jnp.zeros((NRP // NS,), jnp.float32)
    ones = jnp.ones((SUB,), jnp.float32)
    fn = pl.kernel(
        _sc_prepass,
        out_type=jax.ShapeDtypeStruct((NC * NRP,), jnp.float32),
        mesh=_sc_mesh(),
        scratch_types=[
            pltpu.VMEM((NSUB, SUB), jnp.int32),
            pltpu.VMEM((NSUB, SUB), jnp.int32),
            pltpu.VMEM((SUB,), jnp.float32),
            pltpu.VMEM((NRP // NS,), jnp.float32),
            pltpu.VMEM_SHARED((NRP,), jnp.float32),
            pltpu.SemaphoreType.DMA,
        ],
        compiler_params=_sc_params(),
    )
    return fn(dstp, typp, zeros, ones)


def _make_edge_pass(use_rc, mega):
    nmega = NSUB // mega

    def body(table_hbm, srcp, typp, dstp, w_hbm, zrows_hbm, part_hbm,
             *scratch):
        if use_rc:
            (ab, bb, cbuf, wsb, rcb, rows0, rows1, rcsp, acc,
             sem0, sem1, wsem) = scratch
        else:
            (ab, cbuf, wsb, rows0, rows1, acc, sem0, sem1) = scratch
        c = lax.axis_index("c")
        s = lax.axis_index("s")
        w = c * NS + s
        rows_per_tile = NPAD // NS
        pltpu.sync_copy(zrows_hbm, rows0)
        for k in range(rows_per_tile // SUB):
            pltpu.sync_copy(rows0, acc.at[pl.ds(s * rows_per_tile + k * SUB,
                                                SUB)])
        if use_rc:
            quart = NRP // NS // 4
            for h in range(4):
                off = s * (NRP // NS) + h * quart
                pltpu.sync_copy(w_hbm.at[pl.ds(off, quart)], rcb)
                pltpu.sync_copy(rcb, rcsp.at[pl.ds(off, quart)])
        plsc.subcore_barrier()

        def gather(j, rbuf, sem):
            pltpu.async_copy(table_hbm.at[ab.at[j]], rbuf, sem)

        def gwait(j, rbuf, sem):
            pltpu.make_async_copy(table_hbm.at[ab.at[j]], rbuf, sem).wait()

        def process(j, rbuf):
            @plsc.parallel_loop(0, SUB, unroll=2)
            def _scale(r):
                wv = plsc.load_gather(
                    wsb, [jnp.full((16,), j, jnp.int32),
                          jnp.full((16,), r, jnp.int32)])
                for c8 in range(D // 16):
                    sl = pl.ds(c8 * 16, 16)
                    rbuf[r, sl] = rbuf[r, sl] * wv

            pltpu.sync_copy(rbuf, acc.at[cbuf.at[j]], add=True)

        asubs = jnp.minimum(NSUB, jnp.maximum(0, (E // SUB) - w * NSUB))
        for m in range(nmega):
            pairs = jnp.clip((asubs - m * mega) // 2, 0, mega // 2)

            @pl.when(pairs > 0)
            def _mega():
                md = pl.ds(w * NSUB + m * mega, mega)
                pltpu.sync_copy(srcp.at[md], ab)
                pltpu.sync_copy(dstp.at[md], cbuf)
                if use_rc:
                    pltpu.sync_copy(typp.at[md], bb)

                    @plsc.parallel_loop(0, mega, unroll=2)
                    def mkidx(r):
                        for c8 in range(SUB // 16):
                            sl = pl.ds(c8 * 16, 16)
                            tv = bb[r, sl]
                            ab[r, sl] = ab[r, sl] * R + tv
                            bb[r, sl] = cbuf[r, sl] * R + tv

                    for r in range(mega):
                        pltpu.async_copy(rcsp.at[bb.at[r]], wsb.at[r], wsem)
                    for r in range(mega):
                        pltpu.make_async_copy(rcsp.at[bb.at[r]], wsb.at[r],
                                              wsem).wait()
                else:
                    pltpu.sync_copy(w_hbm.at[md], wsb)

                gather(0, rows0, sem0)

                def pair(k, carry):
                    j0 = 2 * k
                    gather(j0 + 1, rows1, sem1)
                    gwait(j0, rows0, sem0)
                    process(j0, rows0)

                    @pl.when(k < pairs - 1)
                    def _():
                        gather(j0 + 2, rows0, sem0)

                    gwait(j0 + 1, rows1, sem1)
                    process(j0 + 1, rows1)
                    return carry

                lax.fori_loop(0, pairs, pair, 0)
        plsc.subcore_barrier()
        for k in range(rows_per_tile // SUB):
            off = s * rows_per_tile + k * SUB
            pltpu.sync_copy(acc.at[pl.ds(off, SUB)], rows0)
            pltpu.sync_copy(rows0, part_hbm.at[pl.ds(c * NPAD + off, SUB)])

    return body


def _run_edge_pass(table, srcp, typp, dstp, w, use_rc):
    mega = 16 if use_rc else 32
    zrows = jnp.zeros((SUB, D), jnp.float32)
    scratch = [pltpu.VMEM((mega, SUB), jnp.int32)]
    if use_rc:
        scratch.append(pltpu.VMEM((mega, SUB), jnp.int32))
    scratch.append(pltpu.VMEM((mega, SUB), jnp.int32))
    scratch.append(pltpu.VMEM((mega, SUB), jnp.float32))
    if use_rc:
        scratch.append(pltpu.VMEM((NRP // NS // 4,), jnp.float32))
    scratch.append(pltpu.VMEM((SUB, D), jnp.float32))
    scratch.append(pltpu.VMEM((SUB, D), jnp.float32))
    if use_rc:
        scratch.append(pltpu.VMEM_SHARED((NRP,), jnp.float32))
    scratch.append(pltpu.VMEM_SHARED((NPAD, D), jnp.float32))
    scratch.append(pltpu.SemaphoreType.DMA)
    scratch.append(pltpu.SemaphoreType.DMA)
    if use_rc:
        scratch.append(pltpu.SemaphoreType.DMA)
    fn = pl.kernel(
        _make_edge_pass(use_rc, mega),
        out_type=jax.ShapeDtypeStruct((NC * NPAD, D), jnp.float32),
        mesh=_sc_mesh(),
        scratch_types=scratch,
        compiler_params=_sc_params(),
    )
    return fn(table, srcp, typp, dstp, w, zrows).reshape(NC, NPAD, D)


def _mm_kernel(a_ref, b_ref, o_ref):
    o_ref[...] = jnp.dot(a_ref[...], b_ref[...],
                         preferred_element_type=jnp.float32)


def _wflat_rc_kernel(comp_ref, basis_ref, cnt_ref, w_ref, rc_ref):
    w_ref[...] = jnp.dot(comp_ref[...], basis_ref[...],
                         preferred_element_type=jnp.float32)
    c = cnt_ref[0] + cnt_ref[1]
    rc_ref[...] = 1.0 / jnp.maximum(c, 1.0)


def _tc_wflat_rc(comp, basis_flat, cnt2):
    return pl.pallas_call(
        _wflat_rc_kernel,
        grid=(8,),
        in_specs=[
            pl.BlockSpec((R, 30), lambda j: (0, 0)),
            pl.BlockSpec((30, 2048), lambda j: (0, j)),
            pl.BlockSpec((NC, NRP // D // 8, D), lambda j: (0, j, 0)),
        ],
        out_specs=[
            pl.BlockSpec((R, 2048), lambda j: (0, j)),
            pl.BlockSpec((NRP // D // 8, D), lambda j: (j, 0)),
        ],
        out_shape=[
            jax.ShapeDtypeStruct((R, D * D), jnp.float32),
            jax.ShapeDtypeStruct((NRP // D, D), jnp.float32),
        ],
    )(comp, basis_flat, cnt2)


def _tc_table1(x, wall):
    return pl.pallas_call(
        _mm_kernel,
        grid=(5,),
        in_specs=[
            pl.BlockSpec((2000, D), lambda i: (i, 0)),
            pl.BlockSpec((D, R * D), lambda i: (0, 0)),
        ],
        out_specs=pl.BlockSpec((2000, R * D), lambda i: (i, 0)),
        out_shape=jax.ShapeDtypeStruct((N, R * D), jnp.float32),
    )(x, wall)


def _gate1_kernel(x_ref, p_ref, root_ref, b1_ref, gw_ref, gb_ref, o_ref):
    x = x_ref[...]
    hg = (p_ref[0] + p_ref[1]
          + jnp.dot(x, root_ref[...], preferred_element_type=jnp.float32)
          + b1_ref[...])
    z = (jnp.dot(x, gw_ref[:D, :], preferred_element_type=jnp.float32)
         + jnp.dot(hg, gw_ref[D:, :], preferred_element_type=jnp.float32)
         + gb_ref[...])
    a = jax.nn.sigmoid(z)
    o_ref[...] = a * hg + (1.0 - a) * x


def _tc_gate1(x, parts, root1, bias1, g1w, g1b):
    return pl.pallas_call(
        _gate1_kernel,
        grid=(5,),
        in_specs=[
            pl.BlockSpec((2000, D), lambda i: (i, 0)),
            pl.BlockSpec((NC, 2000, D), lambda i: (0, i, 0)),
            pl.BlockSpec((D, D), lambda i: (0, 0)),
            pl.BlockSpec((1, D), lambda i: (0, 0)),
            pl.BlockSpec((2 * D, D), lambda i: (0, 0)),
            pl.BlockSpec((1, D), lambda i: (0, 0)),
        ],
        out_specs=pl.BlockSpec((2000, D), lambda i: (i, 0)),
        out_shape=jax.ShapeDtypeStruct((N, D), jnp.float32),
    )(x, parts, root1, bias1, g1w, g1b)


def _gate2_kernel(h1_ref, a_ref, wrel_ref, brel_ref, wroot_ref, gw_ref,
                  gb_ref, o_ref):
    h1 = h1_ref[...]
    agg = a_ref[0] + a_ref[1]
    hg = (jnp.dot(agg, wrel_ref[...], preferred_element_type=jnp.float32)
          + brel_ref[...]
          + jnp.dot(h1, wroot_ref[...], preferred_element_type=jnp.float32))
    z = (jnp.dot(h1, gw_ref[:D, :], preferred_element_type=jnp.float32)
         + jnp.dot(hg, gw_ref[D:, :], preferred_element_type=jnp.float32)
         + gb_ref[...])
    a = jax.nn.sigmoid(z)
    o_ref[...] = a * hg + (1.0 - a) * h1


def _tc_gate2(h1, parts, wrel, brel, wroot, g2w, g2b):
    return pl.pallas_call(
        _gate2_kernel,
        grid=(5,),
        in_specs=[
            pl.BlockSpec((2000, D), lambda i: (i, 0)),
            pl.BlockSpec((NC, 2000, D), lambda i: (0, i, 0)),
            pl.BlockSpec((D, D), lambda i: (0, 0)),
            pl.BlockSpec((1, D), lambda i: (0, 0)),
            pl.BlockSpec((D, D), lambda i: (0, 0)),
            pl.BlockSpec((2 * D, D), lambda i: (0, 0)),
            pl.BlockSpec((1, D), lambda i: (0, 0)),
        ],
        out_specs=pl.BlockSpec((2000, D), lambda i: (i, 0)),
        out_shape=jax.ShapeDtypeStruct((N, D), jnp.float32),
    )(h1, parts, wrel, brel, wroot, g2w, g2b)


def kernel(node_features, edge_index, edge_norm, edge_type, basis, comp,
           root1, bias1, gc_w_rel, gc_b_rel, gc_w_root,
           gate1_w, gate1_b, gate2_w, gate2_b):
    npad_e = EP - E
    srcp = jnp.concatenate(
        [edge_index[0], jnp.zeros((npad_e,), jnp.int32)]).reshape(MROWS, SUB)
    dstp = jnp.concatenate(
        [edge_index[1],
         N + (jnp.arange(npad_e, dtype=jnp.int32) % (NPAD - N))]
    ).reshape(MROWS, SUB)
    typp = jnp.concatenate(
        [edge_type, jnp.zeros((npad_e,), jnp.int32)]).reshape(MROWS, SUB)
    normp = jnp.concatenate(
        [edge_norm, jnp.zeros((npad_e,), jnp.float32)]).reshape(MROWS, SUB)

    cnt2 = _run_prepass(dstp, typp)

    wflat, rc2 = _tc_wflat_rc(comp, basis.reshape(30, D * D),
                              cnt2.reshape(NC, NRP // D, D))
    wall = wflat.reshape(R, D, D).transpose(1, 0, 2).reshape(D, R * D)
    table1 = _tc_table1(node_features, wall).reshape(N * R, D)
    rc = rc2.reshape(NRP)

    p1 = _run_edge_pass(table1, srcp, typp, dstp, rc, use_rc=True)

    h1 = _tc_gate1(node_features, p1, root1, bias1.reshape(1, D),
                   gate1_w, gate1_b.reshape(1, D))

    p2 = _run_edge_pass(h1, srcp, typp, dstp, normp, use_rc=False)

    h2 = _tc_gate2(h1, p2, gc_w_rel, gc_b_rel.reshape(1, D), gc_w_root,
                   gate2_w, gate2_b.reshape(1, D))
    return h2
